# baseline (speedup 1.0000x reference)
import functools

import jax
import jax.numpy as jnp
from jax.experimental import pallas as pl
from jax.experimental.pallas import tpu as pltpu

_BK = 20000


def _nn_kernel(p_ref, c_ref, val_ref, idx_ref, *, bk, num_rows):
    i = pl.program_id(0)

    @pl.when(i == 0)
    def _init():
        val_ref[...] = jnp.full_like(val_ref, jnp.inf)
        idx_ref[...] = jnp.zeros_like(idx_ref)

    p = p_ref[...]
    c = c_ref[...]
    c2 = jnp.sum(c * c, axis=1, keepdims=True)
    caug = jnp.concatenate([c, c2], axis=1)
    paug = jnp.concatenate(
        [-2.0 * p, jnp.ones((p.shape[0], 1), jnp.float32)], axis=1
    )
    scores = jax.lax.dot_general(
        paug, caug, (((1,), (1,)), ((), ())),
        preferred_element_type=jnp.float32,
    )

    local_min = jnp.min(scores, axis=1, keepdims=True)
    lane_ids = jax.lax.broadcasted_iota(jnp.int32, scores.shape, 1)
    masked = jnp.where(scores == local_min, lane_ids, num_rows)
    local_arg = jnp.min(masked, axis=1, keepdims=True)

    prev_v = val_ref[...]
    prev_i = idx_ref[...]
    better = local_min < prev_v
    val_ref[...] = jnp.where(better, local_min, prev_v)
    idx_ref[...] = jnp.where(better, i * bk + local_arg, prev_i)


def kernel(prompt_embs, clip_embs):
    num_rows, d = clip_embs.shape
    p = prompt_embs.shape[0]
    bk = _BK
    grid = num_rows // bk

    val, idx = pl.pallas_call(
        functools.partial(_nn_kernel, bk=bk, num_rows=num_rows),
        grid=(grid,),
        in_specs=[
            pl.BlockSpec((p, d), lambda i: (0, 0)),
            pl.BlockSpec((bk, d), lambda i: (i, 0)),
        ],
        out_specs=[
            pl.BlockSpec((p, 1), lambda i: (0, 0)),
            pl.BlockSpec((p, 1), lambda i: (0, 0)),
        ],
        out_shape=[
            jax.ShapeDtypeStruct((p, 1), jnp.float32),
            jax.ShapeDtypeStruct((p, 1), jnp.int32),
        ],
    )(prompt_embs, clip_embs)

    ids = idx[:, 0]
    return (prompt_embs, prompt_embs, ids)
